# baseline (speedup 1.0000x reference)
import dataclasses
import functools

import jax
import jax.numpy as jnp
from jax import lax
from jax.experimental import pallas as pl
from jax.experimental.pallas import tpu as pltpu
from jax.experimental.pallas import tpu_sc as plsc

HEIGHT = 10
D = 2048
N_NODES = 2046
NP = 2048
BM = 512
BATCH = 4096
NW = 32
BPW = BATCH // NW
L = 16


def _scores_kernel(x_ref, w_ref, xi_ref, c_ref):
    f = jax.lax.dot_general(
        x_ref[...].astype(jnp.bfloat16), w_ref[...],
        (((1,), (0,)), ((), ())),
        preferred_element_type=jnp.float32)
    fb = f.astype(jnp.bfloat16)
    s = jax.lax.dot_general(
        fb, xi_ref[...], (((1,), (0,)), ((), ())),
        preferred_element_type=jnp.float32)
    r = jnp.concatenate([s[:, 1:], s[:, :1]], axis=1)
    c_ref[...] = (s >= r).astype(jnp.int32)


_SC_MESH = plsc.VectorSubcoreMesh(core_axis_name="c", subcore_axis_name="s")

_SC_CP = pltpu.CompilerParams()
if "needs_layout_passes" in pltpu.CompilerParams.__dataclass_fields__:
    _SC_CP = dataclasses.replace(_SC_CP, needs_layout_passes=False)


@functools.partial(
    pl.kernel,
    mesh=_SC_MESH,
    compiler_params=_SC_CP,
    out_type=jax.ShapeDtypeStruct((NW, HEIGHT, BPW), jnp.int32),
    scratch_types=[
        pltpu.VMEM((L, NP), jnp.int32),
        pltpu.VMEM((HEIGHT, BPW), jnp.int32),
        pltpu.SemaphoreType.DMA,
    ],
)
def _sc_descend(c_hbm, out_hbm, rows_v, path_v, sem):
    wid = lax.axis_index("s") * 2 + lax.axis_index("c")
    base = wid * BPW

    @pl.loop(0, BPW, step=L)
    def _sub(j):
        pltpu.sync_copy(c_hbm.at[pl.ds(base + j, L)], rows_v)
        lanes = lax.iota(jnp.int32, L)
        node = jnp.zeros((L,), jnp.int32)
        for h in range(HEIGHT):
            bit = plsc.load_gather(rows_v, [lanes, 2 * node])
            node = 2 * node + 2 - bit
            path_v[h, pl.ds(j, L)] = node

    pltpu.sync_copy(path_v, out_hbm.at[wid])


def kernel(X, W, Xi):
    batch = X.shape[0]
    wb = W.astype(jnp.bfloat16)
    xib = jnp.pad(Xi.astype(jnp.bfloat16), ((0, 0), (0, NP - N_NODES)))

    c = pl.pallas_call(
        _scores_kernel,
        grid=(batch // BM,),
        in_specs=[
            pl.BlockSpec((BM, D), lambda i: (i, 0)),
            pl.BlockSpec((D, D), lambda i: (0, 0)),
            pl.BlockSpec((D, NP), lambda i: (0, 0)),
        ],
        out_specs=pl.BlockSpec((BM, NP), lambda i: (i, 0)),
        out_shape=jax.ShapeDtypeStruct((batch, NP), jnp.int32),
    )(X, wb, xib)

    paths = _sc_descend(c)
    paths = paths.transpose(0, 2, 1).reshape(batch, HEIGHT)
    root = jnp.zeros((batch, 1), dtype=jnp.int32)
    return jnp.concatenate([root, paths], axis=1)
